# baseline (speedup 1.0000x reference)
import functools

import jax
import jax.numpy as jnp
from jax.experimental import pallas as pl
from jax.experimental.pallas import tpu as pltpu
from jax.experimental.pallas import tpu_sc as plsc


def _round_up(x, m):
    return (x + m - 1) // m * m


def _dot(a, b):
    return jax.lax.dot_general(a, b,
                               dimension_numbers=(((1,), (0,)), ((), ())),
                               preferred_element_type=jnp.float32)


def _proj_body(wl_t, pos_t, hop_t, wout_ref, bo_ref, wl_p, pos_p, hop_p):
    H = wout_ref.shape[1]

    def proj(tab_ref, w0, out_ref, extra=0.0):
        rows = out_ref.shape[0]
        t = tab_ref[...]
        t = jnp.pad(t, ((0, rows - t.shape[0]), (0, 0)))
        out_ref[...] = _dot(t, wout_ref[w0:w0 + H, :]) + extra

    proj(wl_t, H, wl_p, bo_ref[...])
    proj(pos_t, 2 * H, pos_p)
    proj(hop_t, 3 * H, hop_p)


def _sc_gather(table, idx2, n_rows, H, window, off_w):
    mesh = plsc.VectorSubcoreMesh(core_axis_name="core",
                                  subcore_axis_name="subcore")

    def body(tab_hbm, i_hbm, o_hbm):
        def inner(i_vmem, o_vmem):
            pltpu.sync_copy(tab_hbm.at[i_vmem.at[0]], o_vmem)

        pltpu.emit_pipeline(
            inner,
            grid=(n_rows // window,),
            in_specs=[pl.BlockSpec((1, window),
                                   index_map=lambda i: (0, off_w + i))],
            out_specs=[pl.BlockSpec((window, H), index_map=lambda i: (i, 0))],
            core_axis_name=("core", "subcore"),
            dimension_semantics=(pltpu.PARALLEL,),
        )(i_hbm, o_hbm)

    k = pl.kernel(body,
                  out_type=jax.ShapeDtypeStruct((n_rows, H), table.dtype),
                  mesh=mesh)
    return k(table, idx2)


def _dense_body(x_ref, wl_ref, hop_ref, posg_ref, w1_ref, b1_ref, g_ref,
                bt_ref, wout_ref, wlp_ref, hopp_ref, _prev_ref, o_ref):
    x = x_ref[...]
    h = _dot(x, w1_ref[...]) + b1_ref[...]
    havg = jnp.full((h.shape[1], h.shape[1]), 1.0 / h.shape[1], jnp.float32)
    mu = _dot(h, havg)
    var = _dot(h * h, havg) - mu * mu
    hn = (h - mu) * jax.lax.rsqrt(var + 1e-5) * g_ref[...] + bt_ref[...]
    f = jnp.maximum(hn, 0.0)
    y = _dot(f, wout_ref[0:h.shape[1], :])

    def gather_acc(idx_ref, tab_ref):
        idx = idx_ref[0, 0, :]
        rows = tab_ref.shape[0]
        onehot = (idx[:, None] == jax.lax.broadcasted_iota(
            jnp.int32, (idx.shape[0], rows), 1)).astype(jnp.float32)
        return _dot(onehot, tab_ref[...])

    y += gather_acc(wl_ref, wlp_ref)
    y += gather_acc(hop_ref, hopp_ref)
    o_ref[...] = y + posg_ref[...]


def kernel(node_features, wl_labels, positions, hop_distances, W1, b1,
           ln_gamma, ln_beta, wl_table, pos_table, hop_table, W_out, b_out):
    N, D = node_features.shape
    H = W1.shape[1]
    f32 = jnp.float32

    B = 2048
    SC_W = 256
    nb = _round_up(N, B) // B
    weights = (0.03, 0.05, 0.08, 0.12, 0.17, 0.25, 0.30)
    sizes = []
    rem = nb
    for w in weights[:-1]:
        take = max(1, min(rem - (len(weights) - 1 - len(sizes)), round(nb * w)))
        take = max(0, min(rem, take))
        sizes.append(take)
        rem -= take
    sizes.append(rem)
    sizes = [s for s in sizes if s > 0]
    CHUNKS = len(sizes)
    n_pad = nb * B
    assert B % SC_W == 0

    wl_rows = _round_up(wl_table.shape[0], 128)
    pos_rows = _round_up(pos_table.shape[0], 128)
    hop_rows = _round_up(hop_table.shape[0], 128)

    wl_p, pos_p, hop_p = pl.pallas_call(
        _proj_body,
        out_shape=(
            jax.ShapeDtypeStruct((wl_rows, H), f32),
            jax.ShapeDtypeStruct((pos_rows, H), f32),
            jax.ShapeDtypeStruct((hop_rows, H), f32),
        ),
    )(wl_table, pos_table, hop_table, W_out, b_out.reshape(1, H))

    pos_flat = jnp.pad(positions, (0, n_pad - N))
    wl3 = jnp.pad(wl_labels, (0, n_pad - N)).reshape(nb, 1, B)
    hop3 = jnp.pad(hop_distances, (0, n_pad - N)).reshape(nb, 1, B)
    row = lambda a: a.reshape(1, H)

    starts = [sum(sizes[:c]) for c in range(CHUNKS)]
    pos_gaths = [
        _sc_gather(
            pos_p,
            jax.lax.dynamic_slice(pos_flat, (c0 * B,), (sz * B,)).reshape(
                1, sz * B),
            sz * B, H, SC_W, 0)
        for c0, sz in zip(starts, sizes)
    ]

    full = lambda s: pl.BlockSpec(s, lambda i: (0,) * len(s))
    out = None
    for c in range(CHUNKS):
        c0 = starts[c]
        nblk = sizes[c]
        in_specs = [
            pl.BlockSpec((B, D), functools.partial(
                lambda c0, i: (c0 + i, 0), c0)),
            pl.BlockSpec((1, 1, B), functools.partial(
                lambda c0, i: (c0 + i, 0, 0), c0)),
            pl.BlockSpec((1, 1, B), functools.partial(
                lambda c0, i: (c0 + i, 0, 0), c0)),
            pl.BlockSpec((B, H), lambda i: (i, 0)),
            full((D, H)),
            full((1, H)),
            full((1, H)),
            full((1, H)),
            full((4 * H, H)),
            full((wl_rows, H)),
            full((hop_rows, H)),
        ]
        args = [node_features, wl3, hop3, pos_gaths[c], W1, row(b1),
                row(ln_gamma), row(ln_beta), W_out, wl_p, hop_p]
        aliases = {}
        if out is None:
            prev = jnp.zeros((8, H), f32)
            in_specs.append(full((8, H)))
        else:
            prev = out
            in_specs.append(pl.BlockSpec(memory_space=pl.ANY))
            aliases = {11: 0}
        args.append(prev)
        out = pl.pallas_call(
            _dense_body,
            grid=(nblk,),
            in_specs=in_specs,
            out_specs=pl.BlockSpec((B, H), functools.partial(
                lambda c0, i: (c0 + i, 0), c0)),
            out_shape=jax.ShapeDtypeStruct((N, H), f32),
            input_output_aliases=aliases,
            compiler_params=pltpu.CompilerParams(
                dimension_semantics=("parallel",)),
        )(*args)

    return out


# speedup vs baseline: 1.0505x; 1.0505x over previous
import functools

import jax
import jax.numpy as jnp
from jax.experimental import pallas as pl
from jax.experimental.pallas import tpu as pltpu
from jax.experimental.pallas import tpu_sc as plsc


def _round_up(x, m):
    return (x + m - 1) // m * m


def _dot(a, b):
    return jax.lax.dot_general(a, b,
                               dimension_numbers=(((1,), (0,)), ((), ())),
                               preferred_element_type=jnp.float32)


def _proj_body(wl_t, pos_t, hop_t, wout_ref, bo_ref, wl_p, pos_p, hop_p):
    H = wout_ref.shape[1]

    def proj(tab_ref, w0, out_ref, extra=0.0):
        rows = out_ref.shape[0]
        t = tab_ref[...]
        t = jnp.pad(t, ((0, rows - t.shape[0]), (0, 0)))
        out_ref[...] = _dot(t, wout_ref[w0:w0 + H, :]) + extra

    proj(wl_t, H, wl_p, bo_ref[...])
    proj(pos_t, 2 * H, pos_p)
    proj(hop_t, 3 * H, hop_p)


def _sc_gather(table, idx2, n_rows, H, window, off_w):
    mesh = plsc.VectorSubcoreMesh(core_axis_name="core",
                                  subcore_axis_name="subcore")

    def body(tab_hbm, i_hbm, o_hbm):
        def inner(i_vmem, o_vmem):
            pltpu.sync_copy(tab_hbm.at[i_vmem.at[0]], o_vmem)

        pltpu.emit_pipeline(
            inner,
            grid=(n_rows // window,),
            in_specs=[pl.BlockSpec((1, window),
                                   index_map=lambda i: (0, off_w + i))],
            out_specs=[pl.BlockSpec((window, H), index_map=lambda i: (i, 0))],
            core_axis_name=("core", "subcore"),
            dimension_semantics=(pltpu.PARALLEL,),
        )(i_hbm, o_hbm)

    k = pl.kernel(body,
                  out_type=jax.ShapeDtypeStruct((n_rows, H), table.dtype),
                  mesh=mesh)
    return k(table, idx2)


def _dense_body(x_ref, wl_ref, hop_ref, posg_ref, w1_ref, b1_ref, g_ref,
                bt_ref, wout_ref, wlp_ref, hopp_ref, _prev_ref, o_ref):
    x = x_ref[...]
    h = _dot(x, w1_ref[...]) + b1_ref[...]
    havg = jnp.full((h.shape[1], h.shape[1]), 1.0 / h.shape[1], jnp.float32)
    mu = _dot(h, havg)
    var = _dot(h * h, havg) - mu * mu
    hn = (h - mu) * jax.lax.rsqrt(var + 1e-5) * g_ref[...] + bt_ref[...]
    f = jnp.maximum(hn, 0.0)
    y = _dot(f, wout_ref[0:h.shape[1], :])

    def gather_acc(idx_ref, tab_ref):
        idx = idx_ref[0, 0, :]
        rows = tab_ref.shape[0]
        onehot = (idx[:, None] == jax.lax.broadcasted_iota(
            jnp.int32, (idx.shape[0], rows), 1)).astype(jnp.float32)
        return _dot(onehot, tab_ref[...])

    y += gather_acc(wl_ref, wlp_ref)
    y += gather_acc(hop_ref, hopp_ref)
    o_ref[...] = y + posg_ref[...]


def kernel(node_features, wl_labels, positions, hop_distances, W1, b1,
           ln_gamma, ln_beta, wl_table, pos_table, hop_table, W_out, b_out):
    N, D = node_features.shape
    H = W1.shape[1]
    f32 = jnp.float32

    B = 2048
    SC_W = 256
    nb = _round_up(N, B) // B
    weights = (0.05, 0.08, 0.12, 0.15, 0.18, 0.20, 0.22)
    sizes = []
    rem = nb
    for w in weights[:-1]:
        take = max(1, min(rem - (len(weights) - 1 - len(sizes)), round(nb * w)))
        take = max(0, min(rem, take))
        sizes.append(take)
        rem -= take
    sizes.append(rem)
    sizes = [s for s in sizes if s > 0]
    CHUNKS = len(sizes)
    n_pad = nb * B
    assert B % SC_W == 0

    wl_rows = _round_up(wl_table.shape[0], 128)
    pos_rows = _round_up(pos_table.shape[0], 128)
    hop_rows = _round_up(hop_table.shape[0], 128)

    wl_p, pos_p, hop_p = pl.pallas_call(
        _proj_body,
        out_shape=(
            jax.ShapeDtypeStruct((wl_rows, H), f32),
            jax.ShapeDtypeStruct((pos_rows, H), f32),
            jax.ShapeDtypeStruct((hop_rows, H), f32),
        ),
    )(wl_table, pos_table, hop_table, W_out, b_out.reshape(1, H))

    pos_flat = jnp.pad(positions, (0, n_pad - N))
    wl3 = jnp.pad(wl_labels, (0, n_pad - N)).reshape(nb, 1, B)
    hop3 = jnp.pad(hop_distances, (0, n_pad - N)).reshape(nb, 1, B)
    row = lambda a: a.reshape(1, H)

    starts = [sum(sizes[:c]) for c in range(CHUNKS)]
    pos_gaths = [
        _sc_gather(
            pos_p,
            jax.lax.dynamic_slice(pos_flat, (c0 * B,), (sz * B,)).reshape(
                1, sz * B),
            sz * B, H, SC_W, 0)
        for c0, sz in zip(starts, sizes)
    ]

    full = lambda s: pl.BlockSpec(s, lambda i: (0,) * len(s))
    out = None
    for c in range(CHUNKS):
        c0 = starts[c]
        nblk = sizes[c]
        in_specs = [
            pl.BlockSpec((B, D), functools.partial(
                lambda c0, i: (c0 + i, 0), c0)),
            pl.BlockSpec((1, 1, B), functools.partial(
                lambda c0, i: (c0 + i, 0, 0), c0)),
            pl.BlockSpec((1, 1, B), functools.partial(
                lambda c0, i: (c0 + i, 0, 0), c0)),
            pl.BlockSpec((B, H), lambda i: (i, 0)),
            full((D, H)),
            full((1, H)),
            full((1, H)),
            full((1, H)),
            full((4 * H, H)),
            full((wl_rows, H)),
            full((hop_rows, H)),
        ]
        args = [node_features, wl3, hop3, pos_gaths[c], W1, row(b1),
                row(ln_gamma), row(ln_beta), W_out, wl_p, hop_p]
        aliases = {}
        if out is None:
            prev = jnp.zeros((8, H), f32)
            in_specs.append(full((8, H)))
        else:
            prev = out
            in_specs.append(pl.BlockSpec(memory_space=pl.ANY))
            aliases = {11: 0}
        args.append(prev)
        out = pl.pallas_call(
            _dense_body,
            grid=(nblk,),
            in_specs=in_specs,
            out_specs=pl.BlockSpec((B, H), functools.partial(
                lambda c0, i: (c0 + i, 0), c0)),
            out_shape=jax.ShapeDtypeStruct((N, H), f32),
            input_output_aliases=aliases,
            compiler_params=pltpu.CompilerParams(
                dimension_semantics=("parallel",)),
        )(*args)

    return out
